# baseline (speedup 1.0000x reference)
import functools

import jax
import jax.numpy as jnp
from jax import lax
from jax.experimental import pallas as pl
from jax.experimental.pallas import tpu as pltpu
from jax.experimental.pallas import tpu_sc as plsc

F = 26
B = 4096
K = 16
AT_H = 32
NPAIR = F * (F - 1) // 2
NTOT = F * B
SP = 8
RB = B // SP
HASH_TABLE_ROWS = 1000000

_NC = 2
_NS = 16
_NW = _NC * _NS
_ROWS_PER_W = NTOT // _NW


@functools.lru_cache(maxsize=None)
def _make_sc_gather():
    mesh = plsc.VectorSubcoreMesh(core_axis_name="c", subcore_axis_name="s")

    @functools.partial(
        pl.kernel,
        mesh=mesh,
        compiler_params=pltpu.CompilerParams(use_tc_tiling_on_sc=False),
        out_type=[
            jax.ShapeDtypeStruct((NTOT // SP, 128), jnp.float32),
            jax.ShapeDtypeStruct((NTOT,), jnp.float32),
        ],
        scratch_types=[
            pltpu.VMEM((_ROWS_PER_W,), jnp.int32),
            pltpu.VMEM((_ROWS_PER_W, K), jnp.float32),
            pltpu.VMEM((_ROWS_PER_W // SP, 128), jnp.float32),
            pltpu.VMEM((_ROWS_PER_W,), jnp.float32),
            pltpu.SemaphoreType.DMA,
            pltpu.SemaphoreType.DMA,
        ],
    )
    def _sc_gather(emb_hbm, w1_hbm, idx_hbm, v_out, w1_out, idx_v, rows_v,
                   rows2, w1_v, sem_rows, sem_w1):
        wid = lax.axis_index("s") * _NC + lax.axis_index("c")
        base = wid * _ROWS_PER_W
        pltpu.sync_copy(idx_hbm.at[pl.ds(base, _ROWS_PER_W)], idx_v)
        cp_rows = pltpu.async_copy(emb_hbm.at[idx_v], rows_v, sem_rows)
        cp_w1 = pltpu.async_copy(w1_hbm.at[idx_v], w1_v, sem_w1)
        cp_rows.wait()
        cp_w1.wait()

        @pl.loop(0, _ROWS_PER_W // SP)
        def _repack(i):
            for j in range(SP):
                rows2[i, pl.ds(K * j, K)] = rows_v[i * SP + j, :]

        pltpu.sync_copy(
            rows2,
            v_out.at[pl.ds(wid * (_ROWS_PER_W // SP), _ROWS_PER_W // SP)])
        pltpu.sync_copy(w1_v, w1_out.at[pl.ds(base, _ROWS_PER_W)])

    return _sc_gather


def _dense_body(v_ref, w1_ref, wp_ref, bp_ref, hp_ref, ep_ref, pp_ref,
                w0_ref, out_ref):
    rt = v_ref.shape[1]
    v = v_ref[...]

    slabs = []
    for i in range(F - 1):
        ni = F - 1 - i
        vi = v[i]
        rest = v[i + 1:]
        prod = jnp.broadcast_to(vi[None], (ni, rt, 128)) * rest
        slabs.append(prod.reshape(ni * rt, 128))
    vv = jnp.concatenate(slabs, axis=0)

    hid = jnp.maximum(
        jnp.dot(vv, wp_ref[...], preferred_element_type=jnp.float32)
        + bp_ref[...], 0.0)
    sc = jnp.dot(hid, hp_ref[...],
                 preferred_element_type=jnp.float32)
    sexp = jnp.dot(sc, ep_ref[...],
                   preferred_element_type=jnp.float32)
    weighted = vv * sexp
    pool = jnp.sum(weighted.reshape(NPAIR, rt, 128), axis=0)

    at_fm = jnp.dot(pool, pp_ref[...],
                    preferred_element_type=jnp.float32)
    fm1 = jnp.sum(w1_ref[...], axis=0)
    out_ref[...] = jax.nn.sigmoid(at_fm + fm1 + w0_ref[0])


def _dense(v_pk, w1_pk, wp, bp, hp, ep, pp, w0, rt):
    grid = (RB // rt,)
    return pl.pallas_call(
        _dense_body,
        grid=grid,
        in_specs=[
            pl.BlockSpec((F, rt, 128), lambda i: (0, i, 0)),
            pl.BlockSpec((F, rt, SP), lambda i: (0, i, 0)),
            pl.BlockSpec((128, SP * AT_H), lambda i: (0, 0)),
            pl.BlockSpec((1, SP * AT_H), lambda i: (0, 0)),
            pl.BlockSpec((SP * AT_H, SP), lambda i: (0, 0)),
            pl.BlockSpec((SP, 128), lambda i: (0, 0)),
            pl.BlockSpec((128, SP), lambda i: (0, 0)),
            pl.BlockSpec(memory_space=pltpu.SMEM),
        ],
        out_specs=pl.BlockSpec((rt, SP), lambda i: (i, 0)),
        out_shape=jax.ShapeDtypeStruct((RB, SP), jnp.float32),
    )(v_pk, w1_pk, wp, bp, hp, ep, pp, w0)


def kernel(x, emb_v, AT_W, AT_B, h, p, w0, w1):
    idx = x.astype(jnp.int32).reshape(NTOT)
    v_flat, w1_flat = _make_sc_gather()(emb_v, w1.reshape(HASH_TABLE_ROWS),
                                        idx)
    v_pk = v_flat.reshape(F, RB, 128)
    w1_pk = w1_flat.reshape(F, RB, SP)

    eye = jnp.eye(SP, dtype=jnp.float32)
    wp = jnp.kron(eye, AT_W)
    bp = jnp.tile(AT_B, SP).reshape(1, SP * AT_H)
    hp = jnp.kron(eye, h)
    ep = jnp.kron(eye, jnp.ones((1, K), jnp.float32))
    pp = jnp.kron(eye, p)

    out = _dense(v_pk, w1_pk, wp, bp, hp, ep, pp, w0.reshape(1), rt=32)
    return out.reshape(B, 1)


# speedup vs baseline: 1.1947x; 1.1947x over previous
import functools

import jax
import jax.numpy as jnp
from jax import lax
from jax.experimental import pallas as pl
from jax.experimental.pallas import tpu as pltpu
from jax.experimental.pallas import tpu_sc as plsc

F = 26
B = 4096
K = 16
AT_H = 32
NPAIR = F * (F - 1) // 2
NTOT = F * B
SP = 8
RB = B // SP
HASH_TABLE_ROWS = 1000000

_NC = 2
_NS = 16
_NW = _NC * _NS
_ROWS_PER_W = NTOT // _NW
_CH = 64
_NCH = _ROWS_PER_W // _CH


@functools.lru_cache(maxsize=None)
def _make_emb_gather():
    mesh = plsc.VectorSubcoreMesh(core_axis_name="c", subcore_axis_name="s")

    @functools.partial(
        pl.kernel,
        mesh=mesh,
        out_type=jax.ShapeDtypeStruct((NTOT // SP, 128), jnp.float32),
        scratch_types=[
            pltpu.VMEM((_ROWS_PER_W,), jnp.int32),
            pltpu.VMEM((_CH, K), jnp.float32),
            pltpu.VMEM((_ROWS_PER_W // SP, 128), jnp.float32),
            pltpu.SemaphoreType.DMA,
            pltpu.SemaphoreType.DMA,
        ],
    )
    def _emb_gather(emb_hbm, idx_hbm, v_out, idx_v, tmp, rows2, sem, sem_i):
        wid = lax.axis_index("s") * _NC + lax.axis_index("c")
        base = wid * _ROWS_PER_W
        del sem_i
        pltpu.sync_copy(idx_hbm.at[pl.ds(base, _ROWS_PER_W)], idx_v)

        @pl.loop(0, _NCH)
        def _chunk(c):
            cps = []
            for jj in range(_CH // 16):
                vec = idx_v[pl.ds(c * _CH + jj * 16, 16)]
                for t in range(16):
                    j = jj * 16 + t
                    cps.append(
                        pltpu.async_copy(emb_hbm.at[pl.ds(vec[t], 1), :],
                                         tmp.at[pl.ds(j, 1), :], sem))
            for cp in cps:
                cp.wait()
            for j in range(_CH):
                rows2[c * (_CH // SP) + j // SP,
                      pl.ds(K * (j % SP), K)] = tmp[j, :]

        pltpu.sync_copy(
            rows2,
            v_out.at[pl.ds(wid * (_ROWS_PER_W // SP), _ROWS_PER_W // SP)])

    return _emb_gather


@functools.lru_cache(maxsize=None)
def _make_w1_gather():
    mesh = plsc.VectorSubcoreMesh(core_axis_name="c", subcore_axis_name="s")

    @functools.partial(
        pl.kernel,
        mesh=mesh,
        compiler_params=pltpu.CompilerParams(use_tc_tiling_on_sc=False),
        out_type=jax.ShapeDtypeStruct((NTOT,), jnp.float32),
        scratch_types=[
            pltpu.VMEM((_ROWS_PER_W,), jnp.int32),
            pltpu.VMEM((_ROWS_PER_W,), jnp.float32),
            pltpu.SemaphoreType.DMA,
        ],
    )
    def _w1_gather(w1_hbm, idx_hbm, w1_out, idx_v, w1_v, sem):
        wid = lax.axis_index("s") * _NC + lax.axis_index("c")
        base = wid * _ROWS_PER_W
        pltpu.sync_copy(idx_hbm.at[pl.ds(base, _ROWS_PER_W)], idx_v)
        pltpu.async_copy(w1_hbm.at[idx_v], w1_v, sem).wait()
        pltpu.sync_copy(w1_v, w1_out.at[pl.ds(base, _ROWS_PER_W)])

    return _w1_gather


def _dense_body(v_ref, w1_ref, wp_ref, bp_ref, hp_ref, ep_ref, pp_ref,
                w0_ref, out_ref):
    rt = v_ref.shape[1]
    v = v_ref[...]

    slabs = []
    for i in range(F - 1):
        ni = F - 1 - i
        vi = v[i]
        rest = v[i + 1:]
        prod = jnp.broadcast_to(vi[None], (ni, rt, 128)) * rest
        slabs.append(prod.reshape(ni * rt, 128))
    vv = jnp.concatenate(slabs, axis=0)

    hid = jnp.maximum(
        jnp.dot(vv, wp_ref[...], preferred_element_type=jnp.float32)
        + bp_ref[...], 0.0)
    sc = jnp.dot(hid, hp_ref[...], preferred_element_type=jnp.float32)
    sexp = jnp.dot(sc, ep_ref[...], preferred_element_type=jnp.float32)
    weighted = vv * sexp
    pool = jnp.sum(weighted.reshape(NPAIR, rt, 128), axis=0)

    at_fm = jnp.dot(pool, pp_ref[...], preferred_element_type=jnp.float32)
    fm1 = jnp.sum(w1_ref[...], axis=0)
    out_ref[...] = jax.nn.sigmoid(at_fm + fm1 + w0_ref[0])


def _dense(v_pk, w1_pk, wp, bp, hp, ep, pp, w0, rt):
    grid = (RB // rt,)
    return pl.pallas_call(
        _dense_body,
        grid=grid,
        in_specs=[
            pl.BlockSpec((F, rt, 128), lambda i: (0, i, 0)),
            pl.BlockSpec((F, rt, SP), lambda i: (0, i, 0)),
            pl.BlockSpec((128, SP * AT_H), lambda i: (0, 0)),
            pl.BlockSpec((1, SP * AT_H), lambda i: (0, 0)),
            pl.BlockSpec((SP * AT_H, SP), lambda i: (0, 0)),
            pl.BlockSpec((SP, 128), lambda i: (0, 0)),
            pl.BlockSpec((128, SP), lambda i: (0, 0)),
            pl.BlockSpec(memory_space=pltpu.SMEM),
        ],
        out_specs=pl.BlockSpec((rt, SP), lambda i: (i, 0)),
        out_shape=jax.ShapeDtypeStruct((RB, SP), jnp.float32),
    )(v_pk, w1_pk, wp, bp, hp, ep, pp, w0)


def kernel(x, emb_v, AT_W, AT_B, h, p, w0, w1):
    idx = x.astype(jnp.int32).reshape(NTOT)
    v_flat = _make_emb_gather()(emb_v, idx)
    w1_flat = _make_w1_gather()(w1.reshape(HASH_TABLE_ROWS), idx)
    v_pk = v_flat.reshape(F, RB, 128)
    w1_pk = w1_flat.reshape(F, RB, SP)

    eye = jnp.eye(SP, dtype=jnp.float32)
    wp = jnp.kron(eye, AT_W)
    bp = jnp.tile(AT_B, SP).reshape(1, SP * AT_H)
    hp = jnp.kron(eye, h)
    ep = jnp.kron(eye, jnp.ones((1, K), jnp.float32))
    pp = jnp.kron(eye, p)

    out = _dense(v_pk, w1_pk, wp, bp, hp, ep, pp, w0.reshape(1), rt=32)
    return out.reshape(B, 1)
